# baseline (speedup 1.0000x reference)
import jax
import jax.numpy as jnp
from jax import lax
from jax.experimental import pallas as pl
from jax.experimental.pallas import tpu as pltpu
from jax.experimental.pallas import tpu_sc as plsc

_N = 10000
_E = 320000
_D = 128
_R = 3
_G = 100
_NP = 10240
_NC = 2
_NT = 16
_NH = _N // _NC
_NHP = 5120
_CH = 125
_EPT = _E // _NT
_NCHUNK = _EPT // _CH
_NDUMP = 128
_ROWS = _NHP + _NDUMP
_ZPT = _ROWS // _NT
_OPT = _NHP // _NT
_NB = 1000
_NBLK = _N // _NB
_NBP = 1024
_TBLK = _NP // _NBP


def _prep_body(x_ref, wr_ref, tw_ref, tv_ref):
    x = x_ref[...]
    tw_ref[0] = jnp.dot(x, wr_ref[0, 0], preferred_element_type=jnp.float32)
    tv_ref[0] = jnp.dot(x, wr_ref[1, 0], preferred_element_type=jnp.float32)


def _tc_prep(x, wrel2):
    return pl.pallas_call(
        _prep_body,
        grid=(_R, _TBLK),
        in_specs=[
            pl.BlockSpec((_NBP, _D), lambda r, i: (i, 0)),
            pl.BlockSpec((2, 1, _D, _D), lambda r, i: (0, r, 0, 0)),
        ],
        out_specs=[
            pl.BlockSpec((1, _NBP, _D), lambda r, i: (r, i, 0)),
            pl.BlockSpec((1, _NBP, _D), lambda r, i: (r, i, 0)),
        ],
        out_shape=[
            jax.ShapeDtypeStruct((_R, _NP, _D), jnp.float32),
            jax.ShapeDtypeStruct((_R, _NP, _D), jnp.float32),
        ],
    )(x, wrel2)


_EPW = _E // (_NC * _NT)
_PCAP = 6144
_PROW = 48
_PCH = 128
_NRING = 3


def _part_body(dstf, gidf, pgo, plo, dst_v, gid_v, bg0, bl0, bg1, bl1):
    c = lax.axis_index("c")
    s = lax.axis_index("s")
    w = c * _NT + s

    pltpu.sync_copy(dstf.at[pl.ds(w * _EPW, _EPW)], dst_v)
    pltpu.sync_copy(gidf.at[pl.ds(w * _EPW, _EPW)], gid_v)

    dumpv = _NHP + (s % 8) * 16 + lax.iota(jnp.int32, 16)
    zv = jnp.zeros((16,), jnp.int32)

    def fill(i, _):
        bg0[pl.ds(i * 16, 16)] = zv
        bl0[pl.ds(i * 16, 16)] = dumpv
        bg1[pl.ds(i * 16, 16)] = zv
        bl1[pl.ds(i * 16, 16)] = dumpv
        return 0

    lax.fori_loop(0, _PCAP // 16, fill, 0)

    iota1 = lax.iota(jnp.int32, 16) + 1

    def body(k, carry):
        c0, c1 = carry
        d = dst_v[pl.ds(16 * k, 16)]
        g = gid_v[pl.ds(16 * k, 16)]
        m1 = d >= _NH
        m0 = jnp.logical_not(m1)
        loc = jnp.where(m1, d - _NH, d)
        cs0 = plsc.cumsum(m0.astype(jnp.int32))
        cs1 = iota1 - cs0
        p0 = jnp.minimum(c0 + cs0 - 1, _PCAP - 1)
        p1 = jnp.minimum(c1 + cs1 - 1, _PCAP - 1)
        plsc.store_scatter(bg0, [p0], g, mask=m0)
        plsc.store_scatter(bl0, [p0], loc, mask=m0)
        plsc.store_scatter(bg1, [p1], g, mask=m1)
        plsc.store_scatter(bl1, [p1], loc, mask=m1)
        n0 = plsc.all_reduce_population_count(m0)
        return (jnp.minimum(c0 + n0, _PCAP - 16),
                jnp.minimum(c1 + (16 - n0), _PCAP - 16))

    zc = jnp.zeros((16,), jnp.int32)
    lax.fori_loop(0, _EPW // 16, body, (zc, zc))

    pltpu.sync_copy(bg0, pgo.at[pl.ds(w * _PCAP, _PCAP)])
    pltpu.sync_copy(bl0, plo.at[pl.ds(w * _PCAP, _PCAP)])
    pltpu.sync_copy(bg1, pgo.at[pl.ds((32 + w) * _PCAP, _PCAP)])
    pltpu.sync_copy(bl1, plo.at[pl.ds((32 + w) * _PCAP, _PCAP)])


def _sc_partition(dstf, gidf):
    mesh = plsc.VectorSubcoreMesh(core_axis_name="c", subcore_axis_name="s")
    return pl.kernel(
        _part_body,
        out_type=[
            jax.ShapeDtypeStruct((2 * 32 * _PCAP,), jnp.int32),
            jax.ShapeDtypeStruct((2 * 32 * _PCAP,), jnp.int32),
        ],
        mesh=mesh,
        compiler_params=pltpu.CompilerParams(needs_layout_passes=False),
        scratch_types=[
            pltpu.VMEM((_EPW,), jnp.int32),
            pltpu.VMEM((_EPW,), jnp.int32),
            pltpu.VMEM((_PCAP,), jnp.int32),
            pltpu.VMEM((_PCAP,), jnp.int32),
            pltpu.VMEM((_PCAP,), jnp.int32),
            pltpu.VMEM((_PCAP,), jnp.int32),
        ],
    )(dstf, gidf)


def _sc_body(tab, pg4, pl4, zrows, out,
             pgv0, plv0, pgv1, plv1, buf0, buf1, buf2, acc,
             gs0, gs1, gs2, ss0, ss1, ss2):
    c = lax.axis_index("c")
    s = lax.axis_index("s")
    bufs = (buf0, buf1, buf2)
    gsems = (gs0, gs1, gs2)
    ssems = (ss0, ss1, ss2)

    pltpu.sync_copy(zrows, acc.at[pl.ds(s * _ZPT, _ZPT)])
    plsc.subcore_barrier()

    pltpu.sync_copy(pg4.at[c, 2 * s], pgv0)
    pltpu.sync_copy(pl4.at[c, 2 * s], plv0)
    pltpu.sync_copy(pg4.at[c, 2 * s + 1], pgv1)
    pltpu.sync_copy(pl4.at[c, 2 * s + 1], plv1)

    for (pg_v, pl_v) in ((pgv0, plv0), (pgv1, plv1)):

        def gather(j, k):
            pltpu.async_copy(tab.at[pg_v.at[j]], bufs[k], gsems[k])

        def gather_wait(j, k):
            pltpu.make_async_copy(tab.at[pg_v.at[j]], bufs[k],
                                  gsems[k]).wait()

        def scatter(j, k):
            pltpu.async_copy(bufs[k], acc.at[pl_v.at[j]], ssems[k],
                             add=True)

        def scatter_wait(j, k):
            pltpu.make_async_copy(bufs[k], acc.at[pl_v.at[j]],
                                  ssems[k]).wait()

        for k in range(_NRING - 1):
            gather(k, k)

        def body(i, _):
            for k in range(_NRING):
                j = _NRING * i + k
                kn = (k + _NRING - 1) % _NRING
                gather_wait(j, k)
                scatter(j, k)

                @pl.when(j + _NRING - 1 < _PROW)
                def _():
                    @pl.when(j >= 1)
                    def _():
                        scatter_wait(j - 1, kn)

                    gather(j + _NRING - 1, kn)

            return 0

        lax.fori_loop(0, _PROW // _NRING, body, 0)

        for t in range(_NRING):
            j = _PROW - _NRING + t
            scatter_wait(j, j % _NRING)

    plsc.subcore_barrier()
    pltpu.sync_copy(acc.at[pl.ds(s * _OPT, _OPT)],
                    out.at[c, pl.ds(s * _OPT, _OPT)])


def _sc_aggregate(tab, pg4, pl4, zrows):
    mesh = plsc.VectorSubcoreMesh(core_axis_name="c", subcore_axis_name="s")
    return pl.kernel(
        _sc_body,
        out_type=jax.ShapeDtypeStruct((_NC, _NHP, _D), jnp.float32),
        mesh=mesh,
        scratch_types=[
            pltpu.VMEM((_PROW, _PCH), jnp.int32),
            pltpu.VMEM((_PROW, _PCH), jnp.int32),
            pltpu.VMEM((_PROW, _PCH), jnp.int32),
            pltpu.VMEM((_PROW, _PCH), jnp.int32),
            pltpu.VMEM((_PCH, _D), jnp.float32),
            pltpu.VMEM((_PCH, _D), jnp.float32),
            pltpu.VMEM((_PCH, _D), jnp.float32),
            pltpu.VMEM_SHARED((_ROWS, _D), jnp.float32),
            pltpu.SemaphoreType.DMA,
            pltpu.SemaphoreType.DMA,
            pltpu.SemaphoreType.DMA,
            pltpu.SemaphoreType.DMA,
            pltpu.SemaphoreType.DMA,
            pltpu.SemaphoreType.DMA,
        ],
    )(tab, pg4, pl4, zrows)


def _tc_body(x_ref, aggw_ref, aggv_ref, wself_ref, b2_ref, wff2_ref,
             bff_ref, qs_ref, nt_ref, asg_ref, gid_ref, out_ref):
    i = pl.program_id(0)
    x = x_ref[...]

    hw = jnp.dot(x, wself_ref[0], preferred_element_type=jnp.float32)
    hv = jnp.dot(x, wself_ref[1], preferred_element_type=jnp.float32)
    hw = jnp.maximum(hw + aggw_ref[0] + b2_ref[0], 0.0)
    hv = jnp.maximum(hv + aggv_ref[0] + b2_ref[1], 0.0)

    w = jnp.abs(jnp.sum(hw * wff2_ref[0], axis=1, keepdims=True)
                + bff_ref[0, 0])
    v = (jnp.sum(hv * wff2_ref[1], axis=1, keepdims=True)
         + bff_ref[0, 1])

    mask = (nt_ref[...] == 1) & (asg_ref[...] == 1)
    s = jnp.where(mask, w * qs_ref[...] + v, 0.0)

    cols = lax.broadcasted_iota(jnp.int32, (_NB, _D), 1)
    onehot = cols == gid_ref[...]
    part = jnp.sum(jnp.where(onehot, s, 0.0), axis=0, keepdims=True)
    part8 = jnp.broadcast_to(part, (8, _D))

    @pl.when(i == 0)
    def _():
        out_ref[...] = part8

    @pl.when(i > 0)
    def _():
        out_ref[...] += part8


def _tc_finish(x, aggw, aggv, wself2, b2, wff2, bff2, qs2, nt2, asg2, gid2):
    full = lambda shape: pl.BlockSpec(shape, lambda i: (0,) * len(shape))
    agg_spec = pl.BlockSpec((1, _NB, _D), lambda i: (i // 5, i % 5, 0))
    col_spec = pl.BlockSpec((_NB, 1), lambda i: (i, 0))
    return pl.pallas_call(
        _tc_body,
        grid=(_NBLK,),
        in_specs=[
            pl.BlockSpec((_NB, _D), lambda i: (i, 0)),
            agg_spec,
            agg_spec,
            full((2, _D, _D)),
            full((2, 1, _D)),
            full((2, 1, _D)),
            pl.BlockSpec(memory_space=pltpu.SMEM),
            col_spec,
            col_spec,
            col_spec,
            col_spec,
        ],
        out_specs=pl.BlockSpec((8, _D), lambda i: (0, 0)),
        out_shape=jax.ShapeDtypeStruct((8, _D), jnp.float32),
    )(x, aggw, aggv, wself2, b2, wff2, bff2, qs2, nt2, asg2, gid2)


def kernel(node_feature, qs, edge_index, edge_type, node_type, assignment,
           graph_ids, Wself_w, Wrel_w, b_w, Wff_w, bff_w,
           Wself_v, Wrel_v, b_v, Wff_v, bff_v):
    src = edge_index[0]
    dst = edge_index[1].astype(jnp.int32)
    gidf = (edge_type * _NP + src).astype(jnp.int32)
    zrows = jnp.zeros((_ZPT, _D), jnp.float32)

    pgo, plo = _sc_partition(dst, gidf)
    pg4 = pgo.reshape(_NC, 32, _PROW, _PCH)
    pl4 = plo.reshape(_NC, 32, _PROW, _PCH)

    wrel2 = jnp.stack([Wrel_w, Wrel_v])
    tw, tv = _tc_prep(node_feature, wrel2)
    tw = tw.reshape(_R * _NP, _D)
    tv = tv.reshape(_R * _NP, _D)

    aggw = _sc_aggregate(tw, pg4, pl4, zrows)
    aggv = _sc_aggregate(tv, pg4, pl4, zrows)

    wself2 = jnp.stack([Wself_w, Wself_v])
    b2 = jnp.stack([b_w, b_v]).reshape(2, 1, _D)
    wff2 = jnp.stack([Wff_w[:, 0], Wff_v[:, 0]]).reshape(2, 1, _D)
    bff2 = jnp.stack([bff_w[0], bff_v[0]]).reshape(1, 2)

    qs2 = qs.reshape(_N, 1)
    nt2 = node_type.reshape(_N, 1)
    asg2 = assignment.reshape(_N, 1)
    gid2 = graph_ids.reshape(_N, 1)

    out8 = _tc_finish(node_feature, aggw, aggv, wself2, b2, wff2, bff2,
                      qs2, nt2, asg2, gid2)
    return out8[0, :_G]


# speedup vs baseline: 12.2154x; 12.2154x over previous
import jax
import jax.numpy as jnp
from jax import lax
from jax.experimental import pallas as pl
from jax.experimental.pallas import tpu as pltpu
from jax.experimental.pallas import tpu_sc as plsc

_N = 10000
_E = 320000
_D = 128
_R = 3
_G = 100
_NP = 10240
_NC = 2
_NT = 16
_NH = _N // _NC
_NHP = 5120
_CH = 125
_EPT = _E // _NT
_NCHUNK = _EPT // _CH
_NDUMP = 128
_ROWS = _NHP + _NDUMP
_ZPT = _ROWS // _NT
_OPT = _NHP // _NT
_NB = 1000
_NBLK = _N // _NB
_NBP = 1024
_TBLK = _NP // _NBP


def _prep_body(x_ref, wr_ref, tw_ref, tv_ref):
    x = x_ref[...]
    tw_ref[0] = jnp.dot(x, wr_ref[0, 0], preferred_element_type=jnp.float32)
    tv_ref[0] = jnp.dot(x, wr_ref[1, 0], preferred_element_type=jnp.float32)


def _tc_prep(x, wrel2):
    return pl.pallas_call(
        _prep_body,
        grid=(_R, _TBLK),
        in_specs=[
            pl.BlockSpec((_NBP, _D), lambda r, i: (i, 0)),
            pl.BlockSpec((2, 1, _D, _D), lambda r, i: (0, r, 0, 0)),
        ],
        out_specs=[
            pl.BlockSpec((1, _NBP, _D), lambda r, i: (r, i, 0)),
            pl.BlockSpec((1, _NBP, _D), lambda r, i: (r, i, 0)),
        ],
        out_shape=[
            jax.ShapeDtypeStruct((_R, _NP, _D), jnp.float32),
            jax.ShapeDtypeStruct((_R, _NP, _D), jnp.float32),
        ],
    )(x, wrel2)


_NRING = 4
_NSTAGE = 4
_SCHUNK = _NCHUNK // _NSTAGE


def _sc_body(tab, gidx, sidx, zrows, out,
             gidx_v, sidx_v, buf0, buf1, buf2, buf3, acc,
             gs0, gs1, gs2, gs3, ss0, ss1, ss2, ss3):
    c = lax.axis_index("c")
    s = lax.axis_index("s")
    bufs = (buf0, buf1, buf2, buf3)
    gsems = (gs0, gs1, gs2, gs3)
    ssems = (ss0, ss1, ss2, ss3)

    pltpu.sync_copy(zrows, acc.at[pl.ds(s * _ZPT, _ZPT)])
    plsc.subcore_barrier()

    def gather(j, k):
        pltpu.async_copy(tab.at[gidx_v.at[j]], bufs[k], gsems[k])

    def gather_wait(j, k):
        pltpu.make_async_copy(tab.at[gidx_v.at[j]], bufs[k],
                              gsems[k]).wait()

    def scatter(j, k):
        pltpu.async_copy(bufs[k], acc.at[sidx_v.at[j]], ssems[k],
                         add=True)

    def scatter_wait(j, k):
        pltpu.make_async_copy(bufs[k], acc.at[sidx_v.at[j]],
                              ssems[k]).wait()

    for st in range(_NSTAGE):
        pltpu.sync_copy(gidx.at[s, pl.ds(st * _SCHUNK, _SCHUNK)], gidx_v)
        pltpu.sync_copy(sidx.at[c, s, pl.ds(st * _SCHUNK, _SCHUNK)], sidx_v)

        for k in range(_NRING - 1):
            gather(k, k)

        def body(i, _):
            for k in range(_NRING):
                j = _NRING * i + k
                kn = (k + _NRING - 1) % _NRING
                gather_wait(j, k)
                scatter(j, k)

                @pl.when(j + _NRING - 1 < _SCHUNK)
                def _():
                    @pl.when(j >= 1)
                    def _():
                        scatter_wait(j - 1, kn)

                    gather(j + _NRING - 1, kn)

            return 0

        lax.fori_loop(0, _SCHUNK // _NRING, body, 0)

        for t in range(_NRING):
            j = _SCHUNK - _NRING + t
            scatter_wait(j, j % _NRING)

    plsc.subcore_barrier()
    pltpu.sync_copy(acc.at[pl.ds(s * _OPT, _OPT)],
                    out.at[c, pl.ds(s * _OPT, _OPT)])


def _sc_aggregate(tab, gidx3, sidx4, zrows):
    mesh = plsc.VectorSubcoreMesh(core_axis_name="c", subcore_axis_name="s")
    return pl.kernel(
        _sc_body,
        out_type=jax.ShapeDtypeStruct((_NC, _NHP, _D), jnp.float32),
        mesh=mesh,
        scratch_types=[
            pltpu.VMEM((_SCHUNK, _CH), jnp.int32),
            pltpu.VMEM((_SCHUNK, _CH), jnp.int32),
            pltpu.VMEM((_CH, _D), jnp.float32),
            pltpu.VMEM((_CH, _D), jnp.float32),
            pltpu.VMEM((_CH, _D), jnp.float32),
            pltpu.VMEM((_CH, _D), jnp.float32),
            pltpu.VMEM_SHARED((_ROWS, _D), jnp.float32),
            pltpu.SemaphoreType.DMA,
            pltpu.SemaphoreType.DMA,
            pltpu.SemaphoreType.DMA,
            pltpu.SemaphoreType.DMA,
            pltpu.SemaphoreType.DMA,
            pltpu.SemaphoreType.DMA,
            pltpu.SemaphoreType.DMA,
            pltpu.SemaphoreType.DMA,
        ],
    )(tab, gidx3, sidx4, zrows)


def _tc_body(x_ref, aggw_ref, aggv_ref, wself_ref, b2_ref, wff2_ref,
             bff_ref, qs_ref, nt_ref, asg_ref, gid_ref, out_ref):
    i = pl.program_id(0)
    x = x_ref[...]

    hw = jnp.dot(x, wself_ref[0], preferred_element_type=jnp.float32)
    hv = jnp.dot(x, wself_ref[1], preferred_element_type=jnp.float32)
    hw = jnp.maximum(hw + aggw_ref[0] + b2_ref[0], 0.0)
    hv = jnp.maximum(hv + aggv_ref[0] + b2_ref[1], 0.0)

    w = jnp.abs(jnp.sum(hw * wff2_ref[0], axis=1, keepdims=True)
                + bff_ref[0, 0])
    v = (jnp.sum(hv * wff2_ref[1], axis=1, keepdims=True)
         + bff_ref[0, 1])

    mask = (nt_ref[...] == 1) & (asg_ref[...] == 1)
    s = jnp.where(mask, w * qs_ref[...] + v, 0.0)

    cols = lax.broadcasted_iota(jnp.int32, (_NB, _D), 1)
    onehot = cols == gid_ref[...]
    part = jnp.sum(jnp.where(onehot, s, 0.0), axis=0, keepdims=True)
    part8 = jnp.broadcast_to(part, (8, _D))

    @pl.when(i == 0)
    def _():
        out_ref[...] = part8

    @pl.when(i > 0)
    def _():
        out_ref[...] += part8


def _tc_finish(x, aggw, aggv, wself2, b2, wff2, bff2, qs2, nt2, asg2, gid2):
    full = lambda shape: pl.BlockSpec(shape, lambda i: (0,) * len(shape))
    agg_spec = pl.BlockSpec((1, _NB, _D), lambda i: (i // 5, i % 5, 0))
    col_spec = pl.BlockSpec((_NB, 1), lambda i: (i, 0))
    return pl.pallas_call(
        _tc_body,
        grid=(_NBLK,),
        in_specs=[
            pl.BlockSpec((_NB, _D), lambda i: (i, 0)),
            agg_spec,
            agg_spec,
            full((2, _D, _D)),
            full((2, 1, _D)),
            full((2, 1, _D)),
            pl.BlockSpec(memory_space=pltpu.SMEM),
            col_spec,
            col_spec,
            col_spec,
            col_spec,
        ],
        out_specs=pl.BlockSpec((8, _D), lambda i: (0, 0)),
        out_shape=jax.ShapeDtypeStruct((8, _D), jnp.float32),
    )(x, aggw, aggv, wself2, b2, wff2, bff2, qs2, nt2, asg2, gid2)


def kernel(node_feature, qs, edge_index, edge_type, node_type, assignment,
           graph_ids, Wself_w, Wrel_w, b_w, Wff_w, bff_w,
           Wself_v, Wrel_v, b_v, Wff_v, bff_v):
    src = edge_index[0]
    dst = edge_index[1]
    gidx3 = (edge_type * _NP + src).astype(jnp.int32)
    gidx3 = gidx3.reshape(_NT, _NCHUNK, _CH)
    half = dst // _NH
    local = (dst - half * _NH).astype(jnp.int32)
    dump = (_NHP + (dst % _NDUMP)).astype(jnp.int32)
    sidx4 = jnp.stack([jnp.where(half == 0, local, dump),
                       jnp.where(half == 1, local, dump)])
    sidx4 = sidx4.reshape(_NC, _NT, _NCHUNK, _CH)
    zrows = jnp.zeros((_ZPT, _D), jnp.float32)

    wrel2 = jnp.stack([Wrel_w, Wrel_v])
    tw, tv = _tc_prep(node_feature, wrel2)
    tw = tw.reshape(_R * _NP, _D)
    tv = tv.reshape(_R * _NP, _D)

    aggw = _sc_aggregate(tw, gidx3, sidx4, zrows)
    aggv = _sc_aggregate(tv, gidx3, sidx4, zrows)

    wself2 = jnp.stack([Wself_w, Wself_v])
    b2 = jnp.stack([b_w, b_v]).reshape(2, 1, _D)
    wff2 = jnp.stack([Wff_w[:, 0], Wff_v[:, 0]]).reshape(2, 1, _D)
    bff2 = jnp.stack([bff_w[0], bff_v[0]]).reshape(1, 2)

    qs2 = qs.reshape(_N, 1)
    nt2 = node_type.reshape(_N, 1)
    asg2 = assignment.reshape(_N, 1)
    gid2 = graph_ids.reshape(_N, 1)

    out8 = _tc_finish(node_feature, aggw, aggv, wself2, b2, wff2, bff2,
                      qs2, nt2, asg2, gid2)
    return out8[0, :_G]
